# baseline (speedup 1.0000x reference)
import jax
import jax.numpy as jnp
from jax import lax
from jax.experimental import pallas as pl
from jax.experimental.pallas import tpu as pltpu

F32 = jnp.float32
BF16 = jnp.bfloat16
CP = 128
C = 64

NP = 64
PITCH = 72
MARG = 8


def _js_for(ksz, p):
    return [j for j in range(-ksz, ksz + 1)
            if any(0 <= t < ksz for t in (p + 2 * j - 1, p + 2 * j, p + 2 * j + 1))]


def _pack_conv(w, b, p):
    ksz = w.shape[0]
    js = _js_for(ksz, p)
    w64 = w[:, :C, :C]
    wj = jnp.zeros((len(js), CP, CP), F32)
    for jj, j in enumerate(js):
        t = p + 2 * j
        if 0 <= t < ksz:
            wj = wj.at[jj, 0:C, 0:C].set(w64[t]).at[jj, C:CP, C:CP].set(w64[t])
        if 0 <= t - 1 < ksz:
            wj = wj.at[jj, 0:C, C:CP].set(w64[t - 1])
        if 0 <= t + 1 < ksz:
            wj = wj.at[jj, C:CP, 0:C].set(w64[t + 1])
    return wj.astype(BF16), jnp.concatenate([b[:, :C], b[:, :C]], axis=1)


def _make_fused_kernel(bt, js_cb, js_i1, js_i2):
    m_rows = bt * PITCH
    mp = bt * NP

    def _body(pb_ref, w2_ref, b2_ref,
              wc1, bc1, wc2, bc2, wc3, bc3, wc4, bc4,
              wi11, bi11, wi12, bi12, wi13, bi13,
              wi21, bi21, wi22, bi22, wi23, bi23,
              o_ref, scr):
        half = (lax.broadcasted_iota(jnp.int32, (m_rows, CP), 1) >= C)
        s_loc = lax.broadcasted_iota(
            jnp.int32, (bt, PITCH, CP), 1).reshape(m_rows, CP) - 6
        pos2 = 2 * s_loc + half.astype(jnp.int32)
        in_range = pos2 >= 0
        m118 = in_range & (pos2 < 118)
        m119 = in_range & (pos2 < 119)
        m120 = in_range & (pos2 < 120)

        e = jnp.dot(pb_ref[0].reshape(mp, CP), w2_ref[...],
                    preferred_element_type=F32)
        o = jnp.dot(pb_ref[1].reshape(mp, CP), w2_ref[...],
                    preferred_element_type=F32)
        en = pltpu.roll(e, shift=mp - 1, axis=0)
        u = pltpu.roll(e, shift=C, axis=1)
        v = pltpu.roll(en, shift=C, axis=1)
        lane = lax.broadcasted_iota(jnp.int32, (mp, CP), 1)
        x = jnp.where(lane < C, u, v)
        p0 = jnp.maximum(jnp.maximum(e, o), x)
        p0 = jnp.maximum(p0 + b2_ref[...], 0.0)
        half_p = (lane >= C).astype(jnp.int32)
        pos2_p = 2 * lax.broadcasted_iota(
            jnp.int32, (bt, NP, CP), 1).reshape(mp, CP) + half_p
        p0 = jnp.where(pos2_p < 117, p0, 0.0)

        scr[...] = jnp.zeros((MARG + m_rows + MARG, CP), BF16)
        p0b = p0.astype(BF16)
        for i in range(bt):
            scr[MARG + i * PITCH + 6: MARG + i * PITCH + 65, :] = (
                p0b[i * NP: i * NP + 59, :])

        def conv(ws, bias, js, relu):
            acc = jnp.zeros((m_rows, CP), F32)
            for jj, j in enumerate(js):
                lhs = scr[MARG + j: MARG + j + m_rows, :]
                acc = acc + jnp.dot(lhs, ws[jj], preferred_element_type=F32)
            acc = acc + bias[...]
            return jnp.maximum(acc, 0.0) if relu else acc

        def put(val):
            scr[MARG: MARG + m_rows, :] = val.astype(BF16)

        idn = conv(wc4, bc4, js_cb, True)
        put(jnp.where(m118, conv(wc1, bc1, js_cb, True), 0.0))
        put(jnp.where(m119, conv(wc2, bc2, js_cb, True), 0.0))
        t3 = conv(wc3, bc3, js_cb, True)
        y = jnp.where(m118, jnp.maximum(t3 + idn, 0.0), 0.0)
        put(y)

        put(jnp.where(m119, conv(wi11, bi11, js_i1, True), 0.0))
        put(jnp.where(m120, conv(wi12, bi12, js_i1, True), 0.0))
        t3 = conv(wi13, bi13, js_i1, False)
        y = jnp.where(m118, jnp.maximum(t3 + y, 0.0), 0.0)
        put(y)

        put(jnp.where(m119, conv(wi21, bi21, js_i2, True), 0.0))
        put(jnp.where(m120, conv(wi22, bi22, js_i2, True), 0.0))
        t3 = conv(wi23, bi23, js_i2, False)
        y = jnp.where(m118, jnp.maximum(t3 + y, 0.0), 0.0)

        o_ref[...] = y.reshape(bt, PITCH, CP)[:, 6:65, :]
    return _body


def _head_kernel(x_ref, w1_ref, b1_ref, w2_ref, o_ref):
    h = jnp.dot(x_ref[...], w1_ref[...], preferred_element_type=F32)
    h = jnp.maximum(h + b1_ref[...], 0.0)
    o_ref[...] = jnp.dot(h, w2_ref[...], preferred_element_type=F32)[None]


def _full_spec(a):
    nd = a.ndim
    return pl.BlockSpec(a.shape, lambda i, nd=nd: (0,) * nd)


def kernel(x, stem_w, stem_b,
           cb_0, cb_1, cb_2, cb_3, cb_4, cb_5, cb_6, cb_7,
           ib1_0, ib1_1, ib1_2, ib1_3, ib1_4, ib1_5,
           ib2_0, ib2_1, ib2_2, ib2_3, ib2_4, ib2_5,
           fc1_w, fc1_b, fc2_w, fc2_b):
    B, S = x.shape[0], x.shape[2]
    bt = B // 2
    LF = 118

    xp = jnp.pad(x[:, 0, :], ((0, 0), (3, 3)))
    s_idx = jnp.arange(NP)[None, :, None]
    ph = jnp.arange(2)[:, None, None]
    f = jnp.arange(CP)[None, None, :]
    start = jnp.where(f < 48, 8 * s_idx + 2 * ph, 8 * s_idx + 4 + 2 * ph)
    idx = jnp.clip(start + jnp.where(f < 48, f, f - 48), 0, S + 5)
    idx = jnp.where(f < 96, idx, 0)
    patches = jnp.transpose(xp[:, idx], (1, 0, 2, 3)).astype(BF16)

    w2 = (jnp.zeros((CP, CP), F32)
          .at[0:48, 0:C].set(stem_w[:, :C])
          .at[48:96, C:CP].set(stem_w[:, :C])).astype(BF16)
    b2 = jnp.concatenate([stem_b[:, :C], stem_b[:, :C]], axis=1)

    js_cb, js_i1, js_i2 = _js_for(24, 12), _js_for(12, 6), _js_for(6, 3)
    wc1, bc1 = _pack_conv(cb_0, cb_1, 12)
    wc2, bc2 = _pack_conv(cb_2, cb_3, 12)
    wc3, bc3 = _pack_conv(cb_4, cb_5, 12)
    wc4, bc4 = _pack_conv(cb_6, cb_7, 12)
    wi11, bi11 = _pack_conv(ib1_0, ib1_1, 6)
    wi12, bi12 = _pack_conv(ib1_2, ib1_3, 6)
    wi13, bi13 = _pack_conv(ib1_4, ib1_5, 6)
    wi21, bi21 = _pack_conv(ib2_0, ib2_1, 3)
    wi22, bi22 = _pack_conv(ib2_2, ib2_3, 3)
    wi23, bi23 = _pack_conv(ib2_4, ib2_5, 3)

    wbs = [w2, b2, wc1, bc1, wc2, bc2, wc3, bc3, wc4, bc4,
           wi11, bi11, wi12, bi12, wi13, bi13,
           wi21, bi21, wi22, bi22, wi23, bi23]
    out = pl.pallas_call(
        _make_fused_kernel(bt, js_cb, js_i1, js_i2),
        out_shape=jax.ShapeDtypeStruct((B, 59, CP), F32),
        grid=(2,),
        in_specs=[pl.BlockSpec((2, bt, NP, CP), lambda i: (0, i, 0, 0))]
                 + [_full_spec(a) for a in wbs],
        out_specs=pl.BlockSpec((bt, 59, CP), lambda i: (i, 0, 0)),
        scratch_shapes=[pltpu.VMEM((MARG + bt * PITCH + MARG, CP), BF16)],
        compiler_params=pltpu.CompilerParams(dimension_semantics=("parallel",)),
    )(patches, *wbs)

    feat = jnp.transpose(out.reshape(B, 59, 2, C),
                         (0, 3, 1, 2)).reshape(B, C * LF)

    n1h = fc1_w.shape[1] // 2
    partial = pl.pallas_call(
        _head_kernel,
        out_shape=jax.ShapeDtypeStruct((2, B, S), F32),
        grid=(2,),
        in_specs=[pl.BlockSpec((B, C * LF), lambda i: (0, 0)),
                  pl.BlockSpec((C * LF, n1h), lambda i: (0, i)),
                  pl.BlockSpec((1, n1h), lambda i: (0, i)),
                  pl.BlockSpec((n1h, S), lambda i: (i, 0))],
        out_specs=pl.BlockSpec((1, B, S), lambda i: (i, 0, 0)),
        compiler_params=pltpu.CompilerParams(dimension_semantics=("parallel",)),
    )(feat, fc1_w, fc1_b, fc2_w)
    return partial[0] + partial[1] + fc2_b
